# baseline (speedup 1.0000x reference)
import functools

import jax
import jax.numpy as jnp
from jax import lax
from jax.experimental import pallas as pl
from jax.experimental.pallas import tpu as pltpu
from jax.experimental.pallas import tpu_sc as plsc

_NUM_CORES = 2
_NUM_SUBCORES = 16
_NW = _NUM_CORES * _NUM_SUBCORES
_L = 16


def _wid():
    return lax.axis_index("s") * _NUM_CORES + lax.axis_index("c")


def _transpose_table(table_t, v):
    full_t = v // 128
    n_iter = full_t // _NW + 1
    tail = v - 128 * full_t
    mesh = plsc.VectorSubcoreMesh(core_axis_name="c", subcore_axis_name="s")

    @functools.partial(
        pl.kernel,
        mesh=mesh,
        out_type=jax.ShapeDtypeStruct((v, 128), jnp.float32),
        scratch_types=[
            pltpu.VMEM((2, 64, 128), jnp.float32),
            pltpu.VMEM((2, 128, 128), jnp.float32),
            pltpu.SemaphoreType.DMA,
            pltpu.SemaphoreType.DMA,
            pltpu.SemaphoreType.DMA,
            pltpu.SemaphoreType.DMA,
        ],
        compiler_params=pltpu.CompilerParams(needs_layout_passes=False),
    )
    def k(tab_hbm, scr_hbm, in_v, out_v, i0, i1, o0, o1):
        w = _wid()
        in_sem = (i0, i1)
        out_sem = (o0, o1)
        lanes = lax.iota(jnp.int32, _L)

        def stage(t, bb, width):
            for beta in range(8):
                pltpu.async_copy(
                    tab_hbm.at[pl.ds(8 * beta, 8), pl.ds(128 * t, width)],
                    in_v.at[bb, pl.ds(8 * beta, 8), pl.ds(0, width)],
                    in_sem[bb],
                )

        def drain_stage(bb, width):
            for beta in range(8):
                pltpu.make_async_copy(
                    tab_hbm.at[pl.ds(0, 8), pl.ds(0, width)],
                    in_v.at[bb, pl.ds(8 * beta, 8), pl.ds(0, width)],
                    in_sem[bb],
                ).wait()

        def xpose(bb, width):
            for j in range(64):
                col = jnp.full((_L,), j, jnp.int32)
                for g in range(width // _L):
                    vals = in_v[bb, j, pl.ds(_L * g, _L)]
                    plsc.store_scatter(out_v.at[bb], [lanes + _L * g, col], vals)

        def write(t, bb, width):
            pltpu.async_copy(
                out_v.at[bb, pl.ds(0, width)],
                scr_hbm.at[pl.ds(128 * t, width)],
                out_sem[bb],
            )

        def wait_write(bb, width):
            pltpu.make_async_copy(
                out_v.at[bb, pl.ds(0, width)],
                scr_hbm.at[pl.ds(0, width)],
                out_sem[bb],
            ).wait()

        def body(i, p):
            t = w + _NW * i
            t_next = t + _NW

            @pl.when(t_next < full_t)
            def _():
                stage(t_next, 1 - p, 128)

            @pl.when(t < full_t)
            def _():
                drain_stage(p, 128)

                @pl.when(i >= 2)
                def _():
                    wait_write(p, 128)

                xpose(p, 128)
                write(t, p, 128)

        stage(w, 0, 128)

        def body2(i2, _):
            body(2 * i2, 0)
            body(2 * i2 + 1, 1)
            return _

        lax.fori_loop(0, (n_iter + 1) // 2, body2, None)
        wait_write(0, 128)
        wait_write(1, 128)

        if tail:
            @pl.when(w == 0)
            def _():
                for j in range(64):
                    pltpu.async_copy(
                        tab_hbm.at[j, pl.ds(128 * full_t, tail)],
                        in_v.at[0, j, pl.ds(0, tail)],
                        in_sem[0],
                    )
                for j in range(64):
                    pltpu.make_async_copy(
                        tab_hbm.at[0, pl.ds(0, tail)],
                        in_v.at[0, j, pl.ds(0, tail)],
                        in_sem[0],
                    ).wait()
                xpose(0, tail)
                write(full_t, 0, tail)
                wait_write(0, tail)

    return k(table_t)


def _gather(idx_t, scratch, s, n, d):
    w_cols = n // _NW
    mesh = plsc.VectorSubcoreMesh(core_axis_name="c", subcore_axis_name="s")

    @functools.partial(
        pl.kernel,
        mesh=mesh,
        out_type=jax.ShapeDtypeStruct((s, d, n), jnp.float32),
        scratch_types=[
            pltpu.VMEM((s, w_cols), jnp.int32),
            pltpu.VMEM((2, w_cols, 128), jnp.float32),
            pltpu.VMEM((2, d, w_cols), jnp.float32),
            pltpu.SemaphoreType.DMA,
            pltpu.SemaphoreType.DMA,
            pltpu.SemaphoreType.DMA,
            pltpu.SemaphoreType.DMA,
            pltpu.SemaphoreType.DMA,
        ],
        compiler_params=pltpu.CompilerParams(needs_layout_passes=False),
    )
    def k(idx_hbm, scr_hbm, out_hbm, idx_v, g_v, o_v, isem, g0, g1, o0, o1):
        w = _wid()
        base = w * w_cols
        gsem = (g0, g1)
        osem = (o0, o1)
        lanes = lax.iota(jnp.int32, _L)

        for row in range(s):
            pltpu.async_copy(
                idx_hbm.at[row, pl.ds(base, w_cols)], idx_v.at[row], isem
            )
        for row in range(s):
            pltpu.make_async_copy(
                idx_hbm.at[0, pl.ds(0, w_cols)], idx_v.at[row], isem
            ).wait()

        def gather(row, bb):
            pltpu.async_copy(scr_hbm.at[idx_v.at[row]], g_v.at[bb], gsem[bb])

        def wait_gather(bb):
            pltpu.make_async_copy(
                scr_hbm.at[pl.ds(0, w_cols)], g_v.at[bb], gsem[bb]
            ).wait()

        def xpose(bb):
            for c in range(d):
                cs = jnp.full((_L,), c, jnp.int32)
                for m in range(w_cols // _L):
                    vals = plsc.load_gather(g_v.at[bb], [lanes + _L * m, cs])
                    o_v[bb, c, pl.ds(_L * m, _L)] = vals

        def write(row, bb):
            pltpu.async_copy(
                o_v.at[bb], out_hbm.at[row, :, pl.ds(base, w_cols)], osem[bb]
            )

        def wait_write(bb):
            pltpu.make_async_copy(
                o_v.at[bb], out_hbm.at[0, :, pl.ds(0, w_cols)], osem[bb]
            ).wait()

        def half(i, row, bb):
            wait_gather(bb)

            @pl.when(i > 0)
            def _():
                wait_write(bb)
            xpose(bb)

            @pl.when(row + 2 < s)
            def _():
                gather(row + 2, bb)
            write(row, bb)

        gather(0, 0)
        gather(1, 1)

        def body(i, _):
            half(i, 2 * i, 0)
            half(i, 2 * i + 1, 1)
            return _

        lax.fori_loop(0, s // 2, body, None)
        wait_write(0)
        wait_write(1)

    return k(idx_t, scratch)


@functools.partial(jax.jit, static_argnames=("n", "s", "d"))
def _embed(inputs, table, n, s, d):
    v = table.shape[0]
    table_t = jnp.swapaxes(table, 0, 1)
    idx_t = jnp.swapaxes(inputs, 0, 1).astype(jnp.int32)
    scratch = _transpose_table(table_t, v)
    out = _gather(idx_t, scratch, s, n, d)
    return jnp.transpose(out, (2, 0, 1))


def kernel(inputs, table):
    n, s = inputs.shape
    d = table.shape[1]
    return _embed(inputs, table, n, s, d)


# speedup vs baseline: 1.0479x; 1.0479x over previous
import functools

import jax
import jax.numpy as jnp
from jax import lax
from jax.experimental import pallas as pl
from jax.experimental.pallas import tpu as pltpu
from jax.experimental.pallas import tpu_sc as plsc

_NUM_CORES = 2
_NUM_SUBCORES = 16
_NW = _NUM_CORES * _NUM_SUBCORES
_L = 16


def _wid():
    return lax.axis_index("s") * _NUM_CORES + lax.axis_index("c")


def _transpose_table(table_t, v):
    full_t = v // 128
    n_iter = full_t // _NW + 1
    tail = v - 128 * full_t
    mesh = plsc.VectorSubcoreMesh(core_axis_name="c", subcore_axis_name="s")

    @functools.partial(
        pl.kernel,
        mesh=mesh,
        out_type=jax.ShapeDtypeStruct((v, 128), jnp.float32),
        scratch_types=[
            pltpu.VMEM((2, 64, 128), jnp.float32),
            pltpu.VMEM((2, 128, 128), jnp.float32),
            pltpu.SemaphoreType.DMA,
            pltpu.SemaphoreType.DMA,
            pltpu.SemaphoreType.DMA,
            pltpu.SemaphoreType.DMA,
        ],
        compiler_params=pltpu.CompilerParams(needs_layout_passes=False),
    )
    def k(tab_hbm, scr_hbm, in_v, out_v, i0, i1, o0, o1):
        w = _wid()
        in_sem = (i0, i1)
        out_sem = (o0, o1)
        lanes = lax.iota(jnp.int32, _L)

        def stage(t, bb, width):
            for beta in range(8):
                pltpu.async_copy(
                    tab_hbm.at[pl.ds(8 * beta, 8), pl.ds(128 * t, width)],
                    in_v.at[bb, pl.ds(8 * beta, 8), pl.ds(0, width)],
                    in_sem[bb],
                )

        def drain_stage(bb, width):
            for beta in range(8):
                pltpu.make_async_copy(
                    tab_hbm.at[pl.ds(0, 8), pl.ds(0, width)],
                    in_v.at[bb, pl.ds(8 * beta, 8), pl.ds(0, width)],
                    in_sem[bb],
                ).wait()

        def xpose(bb, width):
            def rows8(j8, _):
                j0 = j8 * 8
                for jj in range(8):
                    col = jnp.full((_L,), j0 + jj, jnp.int32)
                    for g in range(width // _L):
                        vals = in_v[bb, j0 + jj, pl.ds(_L * g, _L)]
                        plsc.store_scatter(
                            out_v.at[bb], [lanes + _L * g, col], vals
                        )
                return _

            lax.fori_loop(0, 8, rows8, None)

        def write(t, bb, width):
            pltpu.async_copy(
                out_v.at[bb, pl.ds(0, width)],
                scr_hbm.at[pl.ds(128 * t, width)],
                out_sem[bb],
            )

        def wait_write(bb, width):
            pltpu.make_async_copy(
                out_v.at[bb, pl.ds(0, width)],
                scr_hbm.at[pl.ds(0, width)],
                out_sem[bb],
            ).wait()

        def body(i, p):
            t = w + _NW * i
            t_next = t + _NW

            @pl.when(t_next < full_t)
            def _():
                stage(t_next, 1 - p, 128)

            @pl.when(t < full_t)
            def _():
                drain_stage(p, 128)

                @pl.when(i >= 2)
                def _():
                    wait_write(p, 128)

                xpose(p, 128)
                write(t, p, 128)

        stage(w, 0, 128)

        def body2(i2, _):
            body(2 * i2, 0)
            body(2 * i2 + 1, 1)
            return _

        lax.fori_loop(0, (n_iter + 1) // 2, body2, None)
        wait_write(0, 128)
        wait_write(1, 128)

        if tail:
            @pl.when(w == 0)
            def _():
                for j in range(64):
                    pltpu.async_copy(
                        tab_hbm.at[j, pl.ds(128 * full_t, tail)],
                        in_v.at[0, j, pl.ds(0, tail)],
                        in_sem[0],
                    )
                for j in range(64):
                    pltpu.make_async_copy(
                        tab_hbm.at[0, pl.ds(0, tail)],
                        in_v.at[0, j, pl.ds(0, tail)],
                        in_sem[0],
                    ).wait()
                xpose(0, tail)
                write(full_t, 0, tail)
                wait_write(0, tail)

    return k(table_t)


def _gather(idx_t, scratch, s, n, d):
    w_cols = n // _NW
    mesh = plsc.VectorSubcoreMesh(core_axis_name="c", subcore_axis_name="s")

    @functools.partial(
        pl.kernel,
        mesh=mesh,
        out_type=jax.ShapeDtypeStruct((s, d, n), jnp.float32),
        scratch_types=[
            pltpu.VMEM((s, w_cols), jnp.int32),
            pltpu.VMEM((2, w_cols, 128), jnp.float32),
            pltpu.VMEM((2, d, w_cols), jnp.float32),
            pltpu.SemaphoreType.DMA,
            pltpu.SemaphoreType.DMA,
            pltpu.SemaphoreType.DMA,
            pltpu.SemaphoreType.DMA,
            pltpu.SemaphoreType.DMA,
        ],
        compiler_params=pltpu.CompilerParams(needs_layout_passes=False),
    )
    def k(idx_hbm, scr_hbm, out_hbm, idx_v, g_v, o_v, isem, g0, g1, o0, o1):
        w = _wid()
        base = w * w_cols
        gsem = (g0, g1)
        osem = (o0, o1)
        lanes = lax.iota(jnp.int32, _L)

        for row in range(s):
            pltpu.async_copy(
                idx_hbm.at[row, pl.ds(base, w_cols)], idx_v.at[row], isem
            )
        for row in range(s):
            pltpu.make_async_copy(
                idx_hbm.at[0, pl.ds(0, w_cols)], idx_v.at[row], isem
            ).wait()

        def gather(row, bb):
            pltpu.async_copy(scr_hbm.at[idx_v.at[row]], g_v.at[bb], gsem[bb])

        def wait_gather(bb):
            pltpu.make_async_copy(
                scr_hbm.at[pl.ds(0, w_cols)], g_v.at[bb], gsem[bb]
            ).wait()

        def xpose(bb):
            def rows8(c8, _):
                c0 = c8 * 8
                for cc in range(8):
                    cs = jnp.full((_L,), c0 + cc, jnp.int32)
                    for m in range(w_cols // _L):
                        vals = plsc.load_gather(
                            g_v.at[bb], [lanes + _L * m, cs]
                        )
                        o_v[bb, c0 + cc, pl.ds(_L * m, _L)] = vals
                return _

            lax.fori_loop(0, d // 8, rows8, None)

        def write(row, bb):
            pltpu.async_copy(
                o_v.at[bb], out_hbm.at[row, :, pl.ds(base, w_cols)], osem[bb]
            )

        def wait_write(bb):
            pltpu.make_async_copy(
                o_v.at[bb], out_hbm.at[0, :, pl.ds(0, w_cols)], osem[bb]
            ).wait()

        def half(i, row, bb):
            wait_gather(bb)

            @pl.when(i > 0)
            def _():
                wait_write(bb)
            xpose(bb)

            @pl.when(row + 2 < s)
            def _():
                gather(row + 2, bb)
            write(row, bb)

        gather(0, 0)
        gather(1, 1)

        def body(i, _):
            half(i, 2 * i, 0)
            half(i, 2 * i + 1, 1)
            return _

        lax.fori_loop(0, s // 2, body, None)
        wait_write(0)
        wait_write(1)

    return k(idx_t, scratch)


@functools.partial(jax.jit, static_argnames=("n", "s", "d"))
def _embed(inputs, table, n, s, d):
    v = table.shape[0]
    table_t = jnp.swapaxes(table, 0, 1)
    idx_t = jnp.swapaxes(inputs, 0, 1).astype(jnp.int32)
    scratch = _transpose_table(table_t, v)
    out = _gather(idx_t, scratch, s, n, d)
    return jnp.transpose(out, (2, 0, 1))


def kernel(inputs, table):
    n, s = inputs.shape
    d = table.shape[1]
    return _embed(inputs, table, n, s, d)


# speedup vs baseline: 2.1298x; 2.0325x over previous
import functools

import jax
import jax.numpy as jnp
from jax import lax
from jax.experimental import pallas as pl
from jax.experimental.pallas import tpu as pltpu
from jax.experimental.pallas import tpu_sc as plsc

_NUM_CORES = 2
_NUM_SUBCORES = 16
_NW = _NUM_CORES * _NUM_SUBCORES
_CHUNK = 640


@functools.partial(jax.jit, static_argnames=("b", "d"))
def _sc_gather(flat_idx, table, b, d):
    b_per_w = b // _NW
    n_chunks = b_per_w // _CHUNK
    mesh = plsc.VectorSubcoreMesh(core_axis_name="c", subcore_axis_name="s")

    @functools.partial(
        pl.kernel,
        mesh=mesh,
        out_type=jax.ShapeDtypeStruct((b, d), jnp.float32),
        scratch_types=[
            pltpu.VMEM((b_per_w,), jnp.int32),
            pltpu.VMEM((2, _CHUNK, d), jnp.float32),
            pltpu.SemaphoreType.DMA,
            pltpu.SemaphoreType.DMA,
            pltpu.SemaphoreType.DMA,
            pltpu.SemaphoreType.DMA,
        ],
        compiler_params=pltpu.CompilerParams(use_tc_tiling_on_sc=False),
    )
    def k(idx_hbm, table_hbm, out_hbm, idx_v, rows_v, g0, g1, o0, o1):
        wid = lax.axis_index("s") * _NUM_CORES + lax.axis_index("c")
        base = wid * b_per_w
        gat_sems = (g0, g1)
        out_sems = (o0, o1)
        pltpu.sync_copy(idx_hbm.at[pl.ds(base, b_per_w)], idx_v)
        gathers = [None] * n_chunks
        writes = [None] * n_chunks
        for g in range(n_chunks):
            bb = g & 1
            if g >= 2:
                writes[g - 2].wait()
            gathers[g] = pltpu.async_copy(
                table_hbm.at[idx_v.at[pl.ds(g * _CHUNK, _CHUNK)]],
                rows_v.at[bb],
                gat_sems[bb],
            )
            if g >= 1:
                gathers[g - 1].wait()
                writes[g - 1] = pltpu.async_copy(
                    rows_v.at[1 - bb],
                    out_hbm.at[pl.ds(base + (g - 1) * _CHUNK, _CHUNK)],
                    out_sems[1 - bb],
                )
        last = n_chunks - 1
        gathers[last].wait()
        writes[last] = pltpu.async_copy(
            rows_v.at[last & 1],
            out_hbm.at[pl.ds(base + last * _CHUNK, _CHUNK)],
            out_sems[last & 1],
        )
        writes[last - 1].wait()
        writes[last].wait()

    return k(flat_idx, table)


def kernel(inputs, table):
    n, s = inputs.shape
    d = table.shape[1]
    flat = inputs.T.reshape(n * s).astype(jnp.int32)
    out = _sc_gather(flat, table, n * s, d)
    return out.reshape(s, n, d).transpose(1, 0, 2)
